# baseline (speedup 1.0000x reference)
import jax
import jax.numpy as jnp
from jax.experimental import pallas as pl

NODE_DIM = 256
FINAL_DIM = 1024
MAX_NEIGHBORS = 49
NUM_EDGE_TYPES = 1000
BF = jnp.bfloat16
F32 = jnp.float32


def _dot(a, b):
    return jnp.dot(a, b, preferred_element_type=F32)


def _front_body(qt_ref, ipw1_ref, ipb1_ref, ipw2_ref, skw_ref, preb_ref,
                lng_ref, lnb_ref, cpw1_ref, cpb1_ref, cpw2_ref, cpb2_ref,
                ngw1_ref, ngb1_ref, nnw1_ref, nnb1_ref, nnw2_ref, nnb2_ref,
                epw1t_ref, epb1_ref,
                ne_ref, cf_ref, nn_ref, ngh_ref, cep_ref):
    qt = qt_ref[...]
    avg = (qt[:, 0, :] + qt[:, 1, :] + qt[:, 2, :]) * (1.0 / 3.0)
    avg_b = avg.astype(BF)
    h1 = jnp.maximum(_dot(avg_b, ipw1_ref[...]) + ipb1_ref[...], 0.0)
    proj = _dot(h1.astype(BF), ipw2_ref[...])
    skip = _dot(avg_b, skw_ref[...])
    pre = proj + skip + preb_ref[...]
    mu = jnp.mean(pre, axis=-1, keepdims=True)
    var = jnp.mean((pre - mu) ** 2, axis=-1, keepdims=True)
    ne = (pre - mu) * jax.lax.rsqrt(var + 1e-5) * lng_ref[...] + lnb_ref[...]
    ne_ref[...] = ne
    ne_b = ne.astype(BF)
    ch = jnp.maximum(_dot(ne_b, cpw1_ref[...]) + cpb1_ref[...], 0.0)
    cf = _dot(ch.astype(BF), cpw2_ref[...]) + cpb2_ref[...]
    cf_ref[...] = cf
    ngh = jnp.maximum(_dot(ne_b, ngw1_ref[...]) + ngb1_ref[...], 0.0)
    ngh_ref[...] = ngh.astype(BF)
    nh = jnp.maximum(_dot(ne_b, nnw1_ref[...]) + nnb1_ref[...], 0.0)
    nn_ref[...] = _dot(nh.astype(BF), nnw2_ref[...]) + nnb2_ref[...]
    cep_ref[...] = _dot(cf.astype(BF), epw1t_ref[...]) + epb1_ref[...]


def _neigh_body(ngh_ref, cep_ref, w2_ref, b2_ref, epw1b_ref, nf_ref, eh_ref):
    nf = _dot(ngh_ref[...], w2_ref[...]) + b2_ref[0]
    nf_ref[...] = nf
    eh = jnp.maximum(_dot(nf.astype(BF), epw1b_ref[...]) + cep_ref[...], 0.0)
    eh_ref[...] = eh.astype(BF)


def _edge_body(eh_ref, w_ref, b_ref, out_ref):
    out_ref[...] = _dot(eh_ref[...], w_ref[...]) + b_ref[...]


def kernel(quantized_tokens, quantized_indices, ip_w1, ip_b1, ip_w2, ip_b2,
           skip_w, skip_b, ln_g, ln_b, cp_w1, cp_b1, cp_w2, cp_b2,
           ng_w1, ng_b1, ng_w2, ng_b2, ep_w1, ep_b1, ep_w2, ep_b2,
           nn_w1, nn_b1, nn_w2, nn_b2):
    del quantized_indices
    B = quantized_tokens.shape[0]
    D = NODE_DIM
    N = MAX_NEIGHBORS
    E = NUM_EDGE_TYPES

    row = lambda v: v.reshape(1, -1).astype(F32)
    ipw1 = ip_w1.astype(BF)
    ipw2 = ip_w2.astype(BF)
    skw = skip_w.astype(BF)
    cpw1 = cp_w1.astype(BF)
    cpw2 = cp_w2.astype(BF)
    ngw1 = ng_w1.astype(BF)
    ngw2 = ng_w2.astype(BF)
    epw1t = ep_w1[:D].astype(BF)
    epw1b = ep_w1[D:].astype(BF)
    epw2 = ep_w2.astype(BF)
    nnw1 = nn_w1.astype(BF)
    nnw2 = nn_w2.astype(BF)

    ne, cf, nn_logits, ngh, cep = pl.pallas_call(
        _front_body,
        out_shape=[
            jax.ShapeDtypeStruct((B, D), F32),
            jax.ShapeDtypeStruct((B, D), F32),
            jax.ShapeDtypeStruct((B, N + 1), F32),
            jax.ShapeDtypeStruct((B, 2 * D), BF),
            jax.ShapeDtypeStruct((B, D), F32),
        ],
    )(quantized_tokens, ipw1, row(ip_b1), ipw2, skw, row(ip_b2 + skip_b),
      row(ln_g), row(ln_b), cpw1, row(cp_b1), cpw2, row(cp_b2),
      ngw1, row(ng_b1), nnw1, row(nn_b1), nnw2, row(nn_b2),
      epw1t, row(ep_b1))

    nf_flat, eh_flat = pl.pallas_call(
        _neigh_body,
        grid=(N,),
        in_specs=[
            pl.BlockSpec((B, 2 * D), lambda n: (0, 0)),
            pl.BlockSpec((B, D), lambda n: (0, 0)),
            pl.BlockSpec((2 * D, D), lambda n: (0, n)),
            pl.BlockSpec((1, 1, D), lambda n: (n, 0, 0)),
            pl.BlockSpec((D, D), lambda n: (0, 0)),
        ],
        out_specs=[
            pl.BlockSpec((B, D), lambda n: (0, n)),
            pl.BlockSpec((B, D), lambda n: (0, n)),
        ],
        out_shape=[
            jax.ShapeDtypeStruct((B, N * D), F32),
            jax.ShapeDtypeStruct((B, N * D), BF),
        ],
    )(ngh, cep, ngw2, ng_b2.reshape(N, 1, D).astype(F32), epw1b)

    R = B * N
    RT = R // 16
    etl_flat = pl.pallas_call(
        _edge_body,
        grid=(R // RT,),
        in_specs=[
            pl.BlockSpec((RT, D), lambda i: (i, 0)),
            pl.BlockSpec((D, E), lambda i: (0, 0)),
            pl.BlockSpec((1, E), lambda i: (0, 0)),
        ],
        out_specs=pl.BlockSpec((RT, E), lambda i: (i, 0)),
        out_shape=jax.ShapeDtypeStruct((R, E), F32),
    )(eh_flat.reshape(R, D), epw2, row(ep_b2))

    neighbor_features = nf_flat.reshape(B, N, D)
    edge_type_logits = etl_flat.reshape(B, N, E)
    return (ne, cf, neighbor_features, edge_type_logits, nn_logits)
